# baseline (speedup 1.0000x reference)
import functools

import jax
import jax.numpy as jnp
from jax import lax
from jax.experimental import pallas as pl
from jax.experimental.pallas import tpu as pltpu
from jax.experimental.pallas import tpu_sc as plsc


def _gather_sc_t(idx_t, tblT, b, t, d, num_workers=32):
    ndg_total = d // 8
    slab_rows = 32
    mesh = plsc.VectorSubcoreMesh(core_axis_name="c", subcore_axis_name="s")

    @functools.partial(
        pl.kernel,
        mesh=mesh,
        out_type=jax.ShapeDtypeStruct((t, d, b), jnp.float32),
        compiler_params=pltpu.CompilerParams(needs_layout_passes=False),
        scratch_types=[
            pltpu.VMEM((t, b), jnp.int32),
            pltpu.VMEM((slab_rows, d), jnp.float32),
            pltpu.VMEM((2, 16, b), jnp.float32),
            pltpu.SemaphoreType.DMA((2,)),
        ],
    )
    def k(idx_hbm, tblT_hbm, x_hbm, idx_v, slab_v, stg_v, ssem):
        wid = lax.axis_index("s") * 2 + lax.axis_index("c")
        dg0 = wid * ndg_total // num_workers
        dg1 = (wid + 1) * ndg_total // num_workers
        ndg = dg1 - dg0
        pltpu.sync_copy(idx_hbm, idx_v)
        pltpu.sync_copy(tblT_hbm.at[pl.ds(dg0 * 8, slab_rows)], slab_v)

        full16 = ndg == 4

        def wait_buf(pb):
            @pl.when((pb == 0) | full16)
            def _():
                pltpu.make_async_copy(
                    stg_v.at[pb], x_hbm.at[0, pl.ds(0, 16)], ssem.at[pb]
                ).wait()

            @pl.when((pb == 1) & (~full16))
            def _():
                pltpu.make_async_copy(
                    stg_v.at[pb, pl.ds(0, 8)], x_hbm.at[0, pl.ds(0, 8)],
                    ssem.at[pb],
                ).wait()

        def task(step, carry):
            ti, h = carry
            p = step % 2

            @pl.when(step >= 2)
            def _():
                wait_buf(p)

            rows = [
                jnp.full((16,), h * 16 + jj, jnp.int32) for jj in range(16)
            ]

            @plsc.parallel_loop(0, b, step=16, unroll=2)
            def _(b0):
                idx16 = idx_v[ti, pl.ds(b0, 16)]
                for jj in range(16):
                    vals = plsc.load_gather(slab_v, [rows[jj], idx16])
                    stg_v[p, jj, pl.ds(b0, 16)] = vals

            @pl.when((h == 0) | full16)
            def _():
                pltpu.async_copy(
                    stg_v.at[p],
                    x_hbm.at[ti, pl.ds((dg0 + 2 * h) * 8, 16)],
                    ssem.at[p],
                )

            @pl.when((h == 1) & (~full16))
            def _():
                pltpu.async_copy(
                    stg_v.at[p, pl.ds(0, 8)],
                    x_hbm.at[ti, pl.ds((dg0 + 2) * 8, 8)],
                    ssem.at[p],
                )
            return (
                jnp.where(h == 1, ti + 1, ti),
                jnp.where(h == 1, 0, h + 1),
            )

        n_tasks = t * 2
        lax.fori_loop(
            0, n_tasks, task,
            (jnp.int32(0), jnp.int32(0)), unroll=False,
        )
        wait_buf(0)
        wait_buf(1)

    return k(idx_t, tblT)


def kernel(idx, token_embedding_table):
    b, t = idx.shape
    v, d = token_embedding_table.shape
    idx_t = idx.T
    tblT = token_embedding_table.T
    x = _gather_sc_t(idx_t, tblT, b, t, d)
    return jnp.transpose(x, (2, 0, 1))


# speedup vs baseline: 1.2496x; 1.2496x over previous
import functools

import jax
import jax.numpy as jnp
from jax import lax
from jax.experimental import pallas as pl
from jax.experimental.pallas import tpu as pltpu
from jax.experimental.pallas import tpu_sc as plsc


def _gather_sc_t(idx_t, tblT, b, t, d, num_workers=32):
    ndg_total = d // 8
    slab_rows = 32
    mesh = plsc.VectorSubcoreMesh(core_axis_name="c", subcore_axis_name="s")

    @functools.partial(
        pl.kernel,
        mesh=mesh,
        out_type=jax.ShapeDtypeStruct((t, d, b), jnp.float32),
        compiler_params=pltpu.CompilerParams(needs_layout_passes=False),
        scratch_types=[
            pltpu.VMEM((t, b), jnp.int32),
            pltpu.VMEM((slab_rows, d), jnp.float32),
            pltpu.VMEM((4, 8, b), jnp.float32),
            pltpu.SemaphoreType.DMA((4,)),
        ],
    )
    def k(idx_hbm, tblT_hbm, x_hbm, idx_v, slab_v, stg_v, ssem):
        wid = lax.axis_index("s") * 2 + lax.axis_index("c")
        dg0 = wid * ndg_total // num_workers
        dg1 = (wid + 1) * ndg_total // num_workers
        ndg = dg1 - dg0
        pltpu.sync_copy(idx_hbm, idx_v)
        pltpu.sync_copy(tblT_hbm.at[pl.ds(dg0 * 8, slab_rows)], slab_v)

        def task(step, carry):
            ti, g = carry
            p = step % 4

            @pl.when(step >= 4)
            def _():
                pltpu.make_async_copy(
                    stg_v.at[p], x_hbm.at[0, pl.ds(0, 8)], ssem.at[p]
                ).wait()

            rows = [jnp.full((16,), g * 8 + j, jnp.int32) for j in range(8)]

            @plsc.parallel_loop(0, b, step=16, unroll=4)
            def _(b0):
                idx16 = idx_v[ti, pl.ds(b0, 16)]
                for j in range(8):
                    vals = plsc.load_gather(slab_v, [rows[j], idx16])
                    stg_v[p, j, pl.ds(b0, 16)] = vals
            pltpu.async_copy(
                stg_v.at[p],
                x_hbm.at[ti, pl.ds((dg0 + g) * 8, 8)],
                ssem.at[p],
            )
            wrap = g + 1 == ndg
            return (
                jnp.where(wrap, ti + 1, ti),
                jnp.where(wrap, 0, g + 1),
            )

        n_tasks = t * ndg
        lax.fori_loop(
            0, n_tasks, task,
            (jnp.int32(0), jnp.int32(0)), unroll=False,
        )

        def drain(q, _):
            s0 = n_tasks - 4 + q

            @pl.when(s0 >= 0)
            def _():
                pltpu.make_async_copy(
                    stg_v.at[s0 % 4], x_hbm.at[0, pl.ds(0, 8)],
                    ssem.at[s0 % 4],
                ).wait()
            return 0

        lax.fori_loop(0, 4, drain, 0, unroll=True)

    return k(idx_t, tblT)


def kernel(idx, token_embedding_table):
    b, t = idx.shape
    v, d = token_embedding_table.shape
    idx_t = idx.T
    tblT = token_embedding_table.T
    x = _gather_sc_t(idx_t, tblT, b, t, d)
    return jnp.transpose(x, (2, 0, 1))
